# Initial kernel scaffold; baseline (speedup 1.0000x reference)
import jax
import jax.numpy as jnp
from jax.experimental import pallas as pl


def kernel(x, vectors):
    raise NotImplementedError("write your pallas kernel here")


# speedup vs baseline: 2.4436x; 2.4436x over previous
import functools

import jax
import jax.numpy as jnp
from jax import lax
from jax.experimental import pallas as pl
from jax.experimental.pallas import tpu as pltpu

_N = 65536
_D = 32
_K = 512
_BLK = 2048


def _vq_body(x_ref, v_ref, q_ref, idx_ref, loss_ref):
    xb = x_ref[...]
    v = v_ref[...]
    xv = jnp.dot(xb, v, preferred_element_type=jnp.float32)
    rownorm = jnp.sum(xb * xb, axis=1, keepdims=True)
    vnorm = jnp.sum(v * v, axis=0, keepdims=True)
    d = (rownorm - 2.0 * xv) + vnorm
    m = jnp.min(d, axis=1, keepdims=True)
    iota = lax.broadcasted_iota(jnp.int32, d.shape, 1)
    idx = jnp.min(jnp.where(d == m, iota, _K), axis=1)
    idx_ref[...] = idx[:, None]
    onehot = (iota == idx[:, None]).astype(jnp.float32)
    q_ref[...] = lax.dot_general(
        onehot, v, (((1,), (1,)), ((), ())),
        preferred_element_type=jnp.float32)

    @pl.when(pl.program_id(0) == 0)
    def _():
        loss_ref[0] = 0.0

    loss_ref[0] += jnp.sum(m)


def _vq(x, vectors):
    grid = _N // _BLK
    return pl.pallas_call(
        _vq_body,
        grid=(grid,),
        in_specs=[
            pl.BlockSpec((_BLK, _D), lambda i: (i, 0)),
            pl.BlockSpec((_D, _K), lambda i: (0, 0)),
        ],
        out_specs=[
            pl.BlockSpec((_BLK, _D), lambda i: (i, 0)),
            pl.BlockSpec((_BLK, 1), lambda i: (i, 0)),
            pl.BlockSpec(memory_space=pltpu.SMEM),
        ],
        out_shape=[
            jax.ShapeDtypeStruct((_N, _D), jnp.float32),
            jax.ShapeDtypeStruct((_N, 1), jnp.int32),
            jax.ShapeDtypeStruct((1,), jnp.float32),
        ],
    )(x, vectors)


def kernel(x, vectors):
    q, idx, loss_sum = _vq(x, vectors)
    loss = loss_sum[0] / (_N * _D)
    return (q, loss, loss, idx)
